# baseline (speedup 1.0000x reference)
import functools

import jax
import jax.numpy as jnp
from jax import lax
from jax.experimental import pallas as pl
from jax.experimental.pallas import tpu as pltpu
from jax.experimental.pallas import tpu_sc as plsc

NC, NS, L = 2, 16, 16
NW = NC * NS

BN = 2048


def _leaky(x):
    return jnp.maximum(x, 0.01 * x)


def _sc_transpose(table_t, tail_tbl):
    ED, NA = table_t.shape
    ncols = NA // 128
    tail_start = ncols * 128
    tail_len = NA - tail_start
    base_cols = ncols // NW
    rem = ncols % NW
    CW = 4
    CWR = CW * 128
    NSTAGE = (base_cols + 1 + CW - 1) // CW

    mesh = plsc.VectorSubcoreMesh(
        core_axis_name="c", subcore_axis_name="s", num_cores=NC, num_subcores=NS
    )

    @functools.partial(
        pl.kernel,
        out_type=jax.ShapeDtypeStruct((NA, ED), jnp.float32),
        mesh=mesh,
        scratch_types=[
            pltpu.VMEM((ED, CWR), jnp.float32),
            pltpu.VMEM((CWR, ED), jnp.float32),
            pltpu.VMEM((tail_len, ED), jnp.float32),
        ],
        compiler_params=pltpu.CompilerParams(
            use_tc_tiling_on_sc=True, needs_layout_passes=False
        ),
    )
    def tr_kernel(tt_hbm, tail_hbm, out_hbm, chunk_v, outb_v, tailc_v):
        wid = lax.axis_index("s") * NC + lax.axis_index("c")
        lo_col = wid * base_cols + jnp.minimum(wid, rem)
        iota = lax.iota(jnp.int32, L)

        def stage(s, c):
            col0 = jnp.minimum(lo_col + s * CW, ncols - CW)
            start = pl.multiple_of(col0 * 128, 128)
            pltpu.sync_copy(tt_hbm.at[:, pl.ds(start, CWR)], chunk_v)

            def row(r, cc):
                rr = jnp.full((L,), r, jnp.int32)
                for q in range(ED // L):
                    outb_v[r, pl.ds(q * L, L)] = plsc.load_gather(
                        chunk_v, [iota + q * L, rr])
                return cc

            lax.fori_loop(0, CWR, row, 0)
            pltpu.sync_copy(outb_v, out_hbm.at[pl.ds(start, CWR)])
            return c

        lax.fori_loop(0, NSTAGE, stage, 0)

        @pl.when(wid == NW - 1)
        def _tail():
            pltpu.sync_copy(tail_hbm, tailc_v)
            pltpu.sync_copy(tailc_v, out_hbm.at[pl.ds(tail_start, tail_len)])

    return tr_kernel(table_t, tail_tbl)


def _sc_gather_rows(table, act):
    _, ED = table.shape
    (B,) = act.shape
    rpw = B // NW

    mesh = plsc.VectorSubcoreMesh(
        core_axis_name="c", subcore_axis_name="s", num_cores=NC, num_subcores=NS
    )

    @functools.partial(
        pl.kernel,
        out_type=jax.ShapeDtypeStruct((B, ED), jnp.float32),
        mesh=mesh,
        scratch_types=[
            pltpu.VMEM((rpw,), jnp.int32),
            pltpu.VMEM((rpw, ED), jnp.float32),
            pltpu.SemaphoreType.DMA,
        ],
        compiler_params=pltpu.CompilerParams(use_tc_tiling_on_sc=True),
    )
    def gather_kernel(table_hbm, act_hbm, out_hbm, idx_v, emb_v, sem):
        wid = lax.axis_index("s") * NC + lax.axis_index("c")
        b0 = wid * rpw
        pltpu.sync_copy(act_hbm.at[pl.ds(b0, rpw)], idx_v)

        def fire(g, carry):
            vec = idx_v[pl.ds(g * L, L)]
            for lane in range(L):
                r = vec[lane]
                pltpu.async_copy(table_hbm.at[r], emb_v.at[g * L + lane], sem)
            return carry

        lax.fori_loop(0, rpw // L, fire, 0)
        pltpu.make_async_copy(table_hbm.at[pl.ds(0, rpw)], emb_v, sem).wait()
        pltpu.sync_copy(emb_v, out_hbm.at[pl.ds(b0, rpw)])

    return gather_kernel(table, act)


def _tc_assemble(state, emb_raw):
    B, SD = state.shape
    _, ED = emb_raw.shape
    OD = SD + ED

    def body(st_ref, emb_ref, out_ref):
        out_ref[pl.ds(0, SD), :] = _leaky(st_ref[...].T)
        out_ref[pl.ds(SD, ED), :] = _leaky(emb_ref[...].T)

    return pl.pallas_call(
        body,
        grid=(B // BN,),
        in_specs=[
            pl.BlockSpec((BN, SD), lambda i: (i, 0)),
            pl.BlockSpec((BN, ED), lambda i: (i, 0)),
        ],
        out_specs=pl.BlockSpec((OD, BN), lambda i: (0, i)),
        out_shape=jax.ShapeDtypeStruct((OD, B), jnp.float32),
    )(state, emb_raw)


def kernel(state, action, emb_table):
    act = action.astype(jnp.int32)
    tail_rows = (emb_table.shape[0] // 128) * 128
    table_lin = _sc_transpose(emb_table.T, emb_table[tail_rows:])
    emb_raw = _sc_gather_rows(table_lin, act)
    out_t = _tc_assemble(state, emb_raw)
    return out_t.T


# speedup vs baseline: 4.8306x; 4.8306x over previous
import functools

import jax
import jax.numpy as jnp
from jax import lax
from jax.experimental import pallas as pl
from jax.experimental.pallas import tpu as pltpu
from jax.experimental.pallas import tpu_sc as plsc

NC, NS, L = 2, 16, 16
NW = NC * NS

BN = 2048


def _leaky(x):
    return jnp.maximum(x, 0.01 * x)


def _sc_gather_rows(table, act):
    _, ED = table.shape
    (B,) = act.shape
    rpw = B // NW

    mesh = plsc.VectorSubcoreMesh(
        core_axis_name="c", subcore_axis_name="s", num_cores=NC, num_subcores=NS
    )

    @functools.partial(
        pl.kernel,
        out_type=jax.ShapeDtypeStruct((B, ED), jnp.float32),
        mesh=mesh,
        scratch_types=[
            pltpu.VMEM((rpw,), jnp.int32),
            pltpu.VMEM((rpw, ED), jnp.float32),
            pltpu.SemaphoreType.DMA,
        ],
        compiler_params=pltpu.CompilerParams(use_tc_tiling_on_sc=True),
    )
    def gather_kernel(table_hbm, act_hbm, out_hbm, idx_v, emb_v, sem):
        wid = lax.axis_index("s") * NC + lax.axis_index("c")
        b0 = wid * rpw
        pltpu.sync_copy(act_hbm.at[pl.ds(b0, rpw)], idx_v)

        def fire(g, carry):
            vec = idx_v[pl.ds(g * L, L)]
            for lane in range(L):
                r = vec[lane]
                pltpu.async_copy(table_hbm.at[r], emb_v.at[g * L + lane], sem)
            return carry

        lax.fori_loop(0, rpw // L, fire, 0)
        pltpu.make_async_copy(table_hbm.at[pl.ds(0, rpw)], emb_v, sem).wait()
        pltpu.sync_copy(emb_v, out_hbm.at[pl.ds(b0, rpw)])

    return gather_kernel(table, act)


def _tc_assemble(state, emb_raw):
    B, SD = state.shape
    _, ED = emb_raw.shape
    OD = SD + ED

    def body(st_ref, emb_ref, out_ref):
        out_ref[pl.ds(0, SD), :] = _leaky(st_ref[...].T)
        out_ref[pl.ds(SD, ED), :] = _leaky(emb_ref[...].T)

    return pl.pallas_call(
        body,
        grid=(B // BN,),
        in_specs=[
            pl.BlockSpec((BN, SD), lambda i: (i, 0)),
            pl.BlockSpec((BN, ED), lambda i: (i, 0)),
        ],
        out_specs=pl.BlockSpec((OD, BN), lambda i: (0, i)),
        out_shape=jax.ShapeDtypeStruct((OD, B), jnp.float32),
    )(state, emb_raw)


def kernel(state, action, emb_table):
    act = action.astype(jnp.int32)
    emb_raw = _sc_gather_rows(emb_table, act)
    out_t = _tc_assemble(state, emb_raw)
    return out_t.T
